# Initial kernel scaffold; baseline (speedup 1.0000x reference)
import functools

import jax
import jax.numpy as jnp
from jax import lax
from jax.experimental import pallas as pl
from jax.experimental.pallas import tpu as pltpu
from jax.experimental.pallas import tpu_sc as plsc

NC = 2
NS = 16
NW = NC * NS
CHUNK = 128


def _deg_kernel(NP, CH):
    rows_per_tile = NP // NS
    mesh = plsc.VectorSubcoreMesh(core_axis_name="c", subcore_axis_name="s")

    @functools.partial(
        pl.kernel,
        out_type=jax.ShapeDtypeStruct((NC, NP), jnp.float32),
        mesh=mesh,
        scratch_types=[
            pltpu.VMEM((CH, CHUNK), jnp.int32),
            pltpu.VMEM((CHUNK,), jnp.float32),
            pltpu.VMEM((rows_per_tile,), jnp.float32),
            pltpu.VMEM_SHARED((NP,), jnp.float32),
        ],
    )
    def deg_k(dst_hbm, out_hbm, idx_v, ones_v, z_v, deg_sp):
        cid = lax.axis_index("c")
        sid = lax.axis_index("s")
        wid = cid * NS + sid
        ones16 = jnp.ones((16,), jnp.float32)
        zeros16 = jnp.zeros((16,), jnp.float32)
        for i in range(CHUNK // 16):
            ones_v[pl.ds(i * 16, 16)] = ones16
        for i in range(rows_per_tile // 16):
            z_v[pl.ds(i * 16, 16)] = zeros16
        pltpu.sync_copy(dst_hbm.at[wid], idx_v)
        base = sid * rows_per_tile
        pltpu.sync_copy(z_v, deg_sp.at[pl.ds(base, rows_per_tile)])
        plsc.subcore_barrier()

        def body(j, carry):
            pltpu.sync_copy(ones_v, deg_sp.at[idx_v.at[j]], add=True)
            return carry

        lax.fori_loop(0, CH, body, 0)
        plsc.subcore_barrier()
        pltpu.sync_copy(
            deg_sp.at[pl.ds(base, rows_per_tile)],
            out_hbm.at[cid, pl.ds(base, rows_per_tile)],
        )

    return deg_k


def _msg_kernel(NP, OUT, CH):
    rows_per_tile = NP // NS
    mesh = plsc.VectorSubcoreMesh(core_axis_name="c", subcore_axis_name="s")

    @functools.partial(
        pl.kernel,
        out_type=jax.ShapeDtypeStruct((NC, NP, OUT), jnp.float32),
        mesh=mesh,
        scratch_types=[
            pltpu.VMEM((CH, CHUNK), jnp.int32),
            pltpu.VMEM((CH, CHUNK), jnp.int32),
            pltpu.VMEM((CHUNK, OUT), jnp.float32),
            pltpu.VMEM_SHARED((NP, OUT), jnp.float32),
            pltpu.SemaphoreType.DMA,
        ],
    )
    def msg_k(g_hbm, src_hbm, dst_hbm, out_hbm, src_v, dst_v, rows_v, acc_sp, sem):
        cid = lax.axis_index("c")
        sid = lax.axis_index("s")
        wid = cid * NS + sid
        pltpu.sync_copy(src_hbm.at[wid], src_v)
        pltpu.sync_copy(dst_hbm.at[wid], dst_v)
        base = sid * rows_per_tile
        pltpu.sync_copy(
            g_hbm.at[pl.ds(base, rows_per_tile)],
            acc_sp.at[pl.ds(base, rows_per_tile)],
        )
        plsc.subcore_barrier()

        def body(j, carry):
            pltpu.async_copy(g_hbm.at[src_v.at[j]], rows_v, sem).wait()
            pltpu.sync_copy(rows_v, acc_sp.at[dst_v.at[j]], add=True)
            return carry

        lax.fori_loop(0, CH, body, 0)
        plsc.subcore_barrier()
        pltpu.sync_copy(
            acc_sp.at[pl.ds(base, rows_per_tile)],
            out_hbm.at[cid, pl.ds(base, rows_per_tile)],
        )

    return msg_k


def _k2_body(d0_ref, d1_ref, x_ref, w_ref, g_ref, dis_ref):
    deg = d0_ref[...] + d1_ref[...] + 1.0
    dis = lax.rsqrt(deg)
    h = jnp.dot(x_ref[...], w_ref[...], preferred_element_type=jnp.float32)
    g_ref[...] = h * dis
    dis_ref[...] = dis


def _k4_body(a0_ref, a1_ref, g_ref, dis_ref, b_ref, out_ref):
    out_ref[...] = (a0_ref[...] + a1_ref[...] - g_ref[...]) * dis_ref[...] + b_ref[...]


def kernel(x, edge_index, W, b):
    N, IN = x.shape
    OUT = W.shape[1]
    E = edge_index.shape[1]

    CH = -(-E // (NW * CHUNK))
    E_pad = NW * CHUNK * CH
    NP = -(-(N + 16) // 256) * 256

    src = jnp.concatenate(
        [edge_index[0].astype(jnp.int32), jnp.full((E_pad - E,), N, jnp.int32)]
    ).reshape(NW, CH, CHUNK)
    dst = jnp.concatenate(
        [edge_index[1].astype(jnp.int32), jnp.full((E_pad - E,), N, jnp.int32)]
    ).reshape(NW, CH, CHUNK)
    x_pad = jnp.pad(x, ((0, NP - N), (0, 0)))

    deg_parts = _deg_kernel(NP, CH)(dst)

    BLK = 256
    grid = (NP // BLK,)
    g, dis = pl.pallas_call(
        _k2_body,
        grid=grid,
        in_specs=[
            pl.BlockSpec((BLK, 1), lambda i: (i, 0)),
            pl.BlockSpec((BLK, 1), lambda i: (i, 0)),
            pl.BlockSpec((BLK, IN), lambda i: (i, 0)),
            pl.BlockSpec((IN, OUT), lambda i: (0, 0)),
        ],
        out_specs=[
            pl.BlockSpec((BLK, OUT), lambda i: (i, 0)),
            pl.BlockSpec((BLK, 1), lambda i: (i, 0)),
        ],
        out_shape=[
            jax.ShapeDtypeStruct((NP, OUT), jnp.float32),
            jax.ShapeDtypeStruct((NP, 1), jnp.float32),
        ],
    )(deg_parts[0][:, None], deg_parts[1][:, None], x_pad, W)

    acc = _msg_kernel(NP, OUT, CH)(g, src, dst)

    out_full = pl.pallas_call(
        _k4_body,
        grid=grid,
        in_specs=[
            pl.BlockSpec((BLK, OUT), lambda i: (i, 0)),
            pl.BlockSpec((BLK, OUT), lambda i: (i, 0)),
            pl.BlockSpec((BLK, OUT), lambda i: (i, 0)),
            pl.BlockSpec((BLK, 1), lambda i: (i, 0)),
            pl.BlockSpec((1, OUT), lambda i: (0, 0)),
        ],
        out_specs=pl.BlockSpec((BLK, OUT), lambda i: (i, 0)),
        out_shape=jax.ShapeDtypeStruct((NP, OUT), jnp.float32),
    )(acc[0], acc[1], g, dis, b[None, :])

    return (out_full[:N], 0)


# speedup vs baseline: 35.7438x; 35.7438x over previous
import functools

import jax
import jax.numpy as jnp
from jax import lax
from jax.experimental import pallas as pl
from jax.experimental.pallas import tpu as pltpu
from jax.experimental.pallas import tpu_sc as plsc

NC = 2
NS = 16
NW = NC * NS
CHUNK = 128


def _deg_kernel(NP, CH):
    rows_per_tile = NP // NS
    mesh = plsc.VectorSubcoreMesh(core_axis_name="c", subcore_axis_name="s")

    @functools.partial(
        pl.kernel,
        out_type=jax.ShapeDtypeStruct((NC, NP), jnp.float32),
        mesh=mesh,
        scratch_types=[
            pltpu.VMEM((CH, CHUNK), jnp.int32),
            pltpu.VMEM((CHUNK,), jnp.float32),
            pltpu.VMEM((rows_per_tile,), jnp.float32),
            pltpu.VMEM_SHARED((NP,), jnp.float32),
        ],
        compiler_params=pltpu.CompilerParams(use_tc_tiling_on_sc=False),
    )
    def deg_k(dst_hbm, out_hbm, idx_v, ones_v, z_v, deg_sp):
        cid = lax.axis_index("c")
        sid = lax.axis_index("s")
        wid = cid * NS + sid
        ones16 = jnp.ones((16,), jnp.float32)
        zeros16 = jnp.zeros((16,), jnp.float32)
        for i in range(CHUNK // 16):
            ones_v[pl.ds(i * 16, 16)] = ones16
        for i in range(rows_per_tile // 16):
            z_v[pl.ds(i * 16, 16)] = zeros16
        pltpu.sync_copy(dst_hbm.at[wid], idx_v)
        base = sid * rows_per_tile
        pltpu.sync_copy(z_v, deg_sp.at[pl.ds(base, rows_per_tile)])
        plsc.subcore_barrier()

        def body(j, carry):
            pltpu.sync_copy(ones_v, deg_sp.at[idx_v.at[j]], add=True)
            return carry

        lax.fori_loop(0, CH, body, 0)
        plsc.subcore_barrier()
        pltpu.sync_copy(
            deg_sp.at[pl.ds(base, rows_per_tile)],
            out_hbm.at[cid, pl.ds(base, rows_per_tile)],
        )

    return deg_k


def _msg_kernel(NP, OUT, CH):
    rows_per_tile = NP // NS
    mesh = plsc.VectorSubcoreMesh(core_axis_name="c", subcore_axis_name="s")

    @functools.partial(
        pl.kernel,
        out_type=jax.ShapeDtypeStruct((NC, NP, OUT), jnp.float32),
        mesh=mesh,
        scratch_types=[
            pltpu.VMEM((CH, CHUNK), jnp.int32),
            pltpu.VMEM((CH, CHUNK), jnp.int32),
            pltpu.VMEM((CHUNK, OUT), jnp.float32),
            pltpu.VMEM_SHARED((NP, OUT), jnp.float32),
            pltpu.SemaphoreType.DMA,
        ],
        compiler_params=pltpu.CompilerParams(use_tc_tiling_on_sc=False),
    )
    def msg_k(g_hbm, src_hbm, dst_hbm, out_hbm, src_v, dst_v, rows_v, acc_sp, sem):
        cid = lax.axis_index("c")
        sid = lax.axis_index("s")
        wid = cid * NS + sid
        pltpu.sync_copy(src_hbm.at[wid], src_v)
        pltpu.sync_copy(dst_hbm.at[wid], dst_v)
        base = sid * rows_per_tile
        pltpu.sync_copy(
            g_hbm.at[pl.ds(base, rows_per_tile)],
            acc_sp.at[pl.ds(base, rows_per_tile)],
        )
        plsc.subcore_barrier()

        def body(j, carry):
            pltpu.async_copy(g_hbm.at[src_v.at[j]], rows_v, sem).wait()
            pltpu.sync_copy(rows_v, acc_sp.at[dst_v.at[j]], add=True)
            return carry

        lax.fori_loop(0, CH, body, 0)
        plsc.subcore_barrier()
        pltpu.sync_copy(
            acc_sp.at[pl.ds(base, rows_per_tile)],
            out_hbm.at[cid, pl.ds(base, rows_per_tile)],
        )

    return msg_k


def _k2_body(d0_ref, d1_ref, x_ref, w_ref, g_ref, dis_ref):
    deg = d0_ref[...] + d1_ref[...] + 1.0
    dis = lax.rsqrt(deg)
    h = jnp.dot(x_ref[...], w_ref[...], preferred_element_type=jnp.float32)
    g_ref[...] = h * dis
    dis_ref[...] = dis


def _k4_body(a0_ref, a1_ref, g_ref, dis_ref, b_ref, out_ref):
    out_ref[...] = (a0_ref[...] + a1_ref[...] - g_ref[...]) * dis_ref[...] + b_ref[...]


def kernel(x, edge_index, W, b):
    N, IN = x.shape
    OUT = W.shape[1]
    E = edge_index.shape[1]

    CH = -(-E // (NW * CHUNK))
    E_pad = NW * CHUNK * CH
    NP = -(-(N + 16) // 256) * 256

    src = jnp.concatenate(
        [edge_index[0].astype(jnp.int32), jnp.full((E_pad - E,), N, jnp.int32)]
    ).reshape(NW, CH, CHUNK)
    dst = jnp.concatenate(
        [edge_index[1].astype(jnp.int32), jnp.full((E_pad - E,), N, jnp.int32)]
    ).reshape(NW, CH, CHUNK)
    x_pad = jnp.pad(x, ((0, NP - N), (0, 0)))

    deg_parts = _deg_kernel(NP, CH)(dst)

    BLK = 256
    grid = (NP // BLK,)
    g, dis = pl.pallas_call(
        _k2_body,
        grid=grid,
        in_specs=[
            pl.BlockSpec((BLK, 1), lambda i: (i, 0)),
            pl.BlockSpec((BLK, 1), lambda i: (i, 0)),
            pl.BlockSpec((BLK, IN), lambda i: (i, 0)),
            pl.BlockSpec((IN, OUT), lambda i: (0, 0)),
        ],
        out_specs=[
            pl.BlockSpec((BLK, OUT), lambda i: (i, 0)),
            pl.BlockSpec((BLK, 1), lambda i: (i, 0)),
        ],
        out_shape=[
            jax.ShapeDtypeStruct((NP, OUT), jnp.float32),
            jax.ShapeDtypeStruct((NP, 1), jnp.float32),
        ],
    )(deg_parts[0][:, None], deg_parts[1][:, None], x_pad, W)

    acc = _msg_kernel(NP, OUT, CH)(g, src, dst)

    out_full = pl.pallas_call(
        _k4_body,
        grid=grid,
        in_specs=[
            pl.BlockSpec((BLK, OUT), lambda i: (i, 0)),
            pl.BlockSpec((BLK, OUT), lambda i: (i, 0)),
            pl.BlockSpec((BLK, OUT), lambda i: (i, 0)),
            pl.BlockSpec((BLK, 1), lambda i: (i, 0)),
            pl.BlockSpec((1, OUT), lambda i: (0, 0)),
        ],
        out_specs=pl.BlockSpec((BLK, OUT), lambda i: (i, 0)),
        out_shape=jax.ShapeDtypeStruct((NP, OUT), jnp.float32),
    )(acc[0], acc[1], g, dis, b[None, :])

    return (out_full[:N], 0)


# speedup vs baseline: 69.3918x; 1.9414x over previous
import functools

import jax
import jax.numpy as jnp
from jax import lax
from jax.experimental import pallas as pl
from jax.experimental.pallas import tpu as pltpu
from jax.experimental.pallas import tpu_sc as plsc

NC = 2
NS = 16
NW = NC * NS
CHUNK = 128
NB = 8
LANES = 16


def _rsqrt16(x):
    i = plsc.bitcast(x, jnp.int32)
    i = jnp.int32(0x5F3759DF) - lax.shift_right_arithmetic(i, 1)
    y = plsc.bitcast(i, jnp.float32)
    xh = x * 0.5
    for _ in range(3):
        y = y * (1.5 - xh * y * y)
    return y


def _main_kernel(N, OUT, NCH, NPR):
    NP = NS * NPR
    TAIL = N - (NS - 1) * NPR
    QD = NCH // NS
    RD = NCH % NS
    QM = NCH // NW
    RM = NCH % NW
    mesh = plsc.VectorSubcoreMesh(core_axis_name="c", subcore_axis_name="s")

    @functools.partial(
        pl.kernel,
        out_type=jax.ShapeDtypeStruct((NC, NP, OUT), jnp.float32),
        mesh=mesh,
        scratch_types=[
            pltpu.VMEM((QD + 1, CHUNK), jnp.int32),
            pltpu.VMEM((QM + 1, CHUNK), jnp.int32),
            pltpu.VMEM((QM + 1, CHUNK), jnp.int32),
            pltpu.VMEM((CHUNK,), jnp.float32),
            pltpu.VMEM((NB, CHUNK, OUT), jnp.float32),
            pltpu.VMEM((NPR,), jnp.float32),
            pltpu.VMEM((NPR,), jnp.float32),
            pltpu.VMEM((NPR, OUT), jnp.float32),
            pltpu.VMEM((NPR, OUT), jnp.float32),
            pltpu.VMEM((OUT,), jnp.float32),
            pltpu.VMEM_SHARED((NP,), jnp.float32),
            pltpu.VMEM_SHARED((NP, OUT), jnp.float32),
            pltpu.VMEM_SHARED((NP, OUT), jnp.float32),
            [pltpu.SemaphoreType.DMA] * NB,
            [pltpu.SemaphoreType.DMA] * NB,
        ],
        compiler_params=pltpu.CompilerParams(
            use_tc_tiling_on_sc=False, needs_layout_passes=False
        ),
    )
    def main_k(h_hbm, ei_hbm, b_hbm, out_hbm, didx_v, msrc_v, mdst_v, ones_v,
               rows_v, degb, disb, hbuf, abuf, bbuf, deg_sp, g_sp, acc_sp,
               gsems, ssems):
        cid = lax.axis_index("c")
        sid = lax.axis_index("s")
        wid = cid * NS + sid
        base = sid * NPR

        ones16 = jnp.ones((LANES,), jnp.float32)
        zeros16 = jnp.zeros((LANES,), jnp.float32)
        for i in range(CHUNK // LANES):
            ones_v[pl.ds(i * LANES, LANES)] = ones16
        for i in range(NPR // LANES):
            degb[pl.ds(i * LANES, LANES)] = zeros16

        dbase = sid * QD
        dcnt = QD + (sid < RD).astype(jnp.int32)
        pltpu.sync_copy(ei_hbm.at[1, pl.ds(dbase, QD)], didx_v.at[pl.ds(0, QD)])

        @pl.when(sid < RD)
        def _():
            pltpu.sync_copy(
                ei_hbm.at[1, pl.ds(NS * QD + sid, 1)], didx_v.at[pl.ds(QD, 1)]
            )

        pltpu.sync_copy(degb, deg_sp.at[pl.ds(base, NPR)])
        plsc.subcore_barrier()

        def deg_grp(i, c):
            j0 = i * NB
            descs = [
                pltpu.async_copy(
                    ones_v, deg_sp.at[didx_v.at[j0 + k]], ssems[k], add=True
                )
                for k in range(NB)
            ]
            for d in descs:
                d.wait()
            return c

        lax.fori_loop(0, QD // NB, deg_grp, 0)
        for k in range(QD + 1 - (QD // NB) * NB):
            j = (QD // NB) * NB + k

            @pl.when(j < dcnt)
            def _():
                pltpu.sync_copy(ones_v, deg_sp.at[didx_v.at[j]], add=True)

        mbase = wid * QM
        mcnt = QM + (wid < RM).astype(jnp.int32)
        pltpu.sync_copy(ei_hbm.at[0, pl.ds(mbase, QM)], msrc_v.at[pl.ds(0, QM)])
        pltpu.sync_copy(ei_hbm.at[1, pl.ds(mbase, QM)], mdst_v.at[pl.ds(0, QM)])

        @pl.when(wid < RM)
        def _():
            pltpu.sync_copy(
                ei_hbm.at[0, pl.ds(NW * QM + wid, 1)], msrc_v.at[pl.ds(QM, 1)]
            )
            pltpu.sync_copy(
                ei_hbm.at[1, pl.ds(NW * QM + wid, 1)], mdst_v.at[pl.ds(QM, 1)]
            )

        pltpu.sync_copy(b_hbm, bbuf)
        plsc.subcore_barrier()

        pltpu.sync_copy(deg_sp.at[pl.ds(base, NPR)], degb)

        @pl.when(sid < NS - 1)
        def _():
            pltpu.sync_copy(h_hbm.at[pl.ds(base, NPR)], hbuf)

        @pl.when(sid == NS - 1)
        def _():
            pltpu.sync_copy(
                h_hbm.at[pl.ds(base, TAIL)], hbuf.at[pl.ds(0, TAIL)]
            )
            for i in range(TAIL, NPR):
                hbuf[i, :] = zeros16

        def dis_loop(i, c):
            off = pl.multiple_of(i * LANES, LANES)
            d = degb[pl.ds(off, LANES)] + 1.0
            disb[pl.ds(off, LANES)] = _rsqrt16(d)
            return c

        lax.fori_loop(0, NPR // LANES, dis_loop, 0)

        def scale_loop(i, c):
            off = pl.multiple_of(i * LANES, LANES)
            dvec = disb[pl.ds(off, LANES)]
            for r in range(LANES):
                hbuf[off + r, :] = hbuf[off + r, :] * dvec[r]
            return c

        lax.fori_loop(0, NPR // LANES, scale_loop, 0)
        pltpu.sync_copy(hbuf, g_sp.at[pl.ds(base, NPR)])
        pltpu.sync_copy(hbuf, acc_sp.at[pl.ds(base, NPR)])
        plsc.subcore_barrier()

        def msg_grp(i, c):
            j0 = i * NB
            gd = [
                pltpu.async_copy(
                    g_sp.at[msrc_v.at[j0 + k]], rows_v.at[k], gsems[k]
                )
                for k in range(NB)
            ]
            sd = []
            for k in range(NB):
                gd[k].wait()
                sd.append(
                    pltpu.async_copy(
                        rows_v.at[k], acc_sp.at[mdst_v.at[j0 + k]], ssems[k],
                        add=True,
                    )
                )
            for d in sd:
                d.wait()
            return c

        lax.fori_loop(0, QM // NB, msg_grp, 0)
        for k in range(QM + 1 - (QM // NB) * NB):
            j = (QM // NB) * NB + k

            @pl.when(j < mcnt)
            def _():
                pltpu.async_copy(
                    g_sp.at[msrc_v.at[j]], rows_v.at[0], gsems[0]
                ).wait()
                pltpu.sync_copy(
                    rows_v.at[0], acc_sp.at[mdst_v.at[j]], add=True
                )

        plsc.subcore_barrier()

        pltpu.sync_copy(acc_sp.at[pl.ds(base, NPR)], abuf)
        sel0 = lax.select(cid == 0, 1.0, 0.0)
        bvec = bbuf[...]

        def fin_loop(i, c):
            off = pl.multiple_of(i * LANES, LANES)
            dvec = disb[pl.ds(off, LANES)]
            for r in range(LANES):
                row = abuf[off + r, :] - (1.0 - sel0) * hbuf[off + r, :]
                abuf[off + r, :] = row * dvec[r] + sel0 * bvec
            return c

        lax.fori_loop(0, NPR // LANES, fin_loop, 0)
        pltpu.sync_copy(abuf, out_hbm.at[cid, pl.ds(base, NPR)])

    return main_k


def _pre_body(x_ref, w_ref, h_ref):
    h_ref[...] = jnp.dot(
        x_ref[...], w_ref[...], preferred_element_type=jnp.float32
    )


def _post_body(a_ref, b_ref, o_ref):
    o_ref[...] = a_ref[...] + b_ref[...]


def kernel(x, edge_index, W, b):
    N, IN = x.shape
    OUT = W.shape[1]
    E = edge_index.shape[1]

    ei = edge_index.astype(jnp.int32)
    if E % CHUNK:
        pad = CHUNK - E % CHUNK
        ei = jnp.concatenate([ei, jnp.full((2, pad), N, jnp.int32)], axis=1)
    NCH = ei.shape[1] // CHUNK
    ei3 = ei.reshape(2, NCH, CHUNK)

    NPR = -(-(N + 1) // (NS * LANES)) * LANES
    NP = NS * NPR

    BLK = 400 if N % 400 == 0 else 8
    h = pl.pallas_call(
        _pre_body,
        grid=(N // BLK,),
        in_specs=[
            pl.BlockSpec((BLK, IN), lambda i: (i, 0)),
            pl.BlockSpec((IN, OUT), lambda i: (0, 0)),
        ],
        out_specs=pl.BlockSpec((BLK, OUT), lambda i: (i, 0)),
        out_shape=jax.ShapeDtypeStruct((N, OUT), jnp.float32),
    )(x, W)

    o = _main_kernel(N, OUT, NCH, NPR)(h, ei3, b)

    PBLK = NPR
    out_full = pl.pallas_call(
        _post_body,
        grid=(NP // PBLK,),
        in_specs=[
            pl.BlockSpec((PBLK, OUT), lambda i: (i, 0)),
            pl.BlockSpec((PBLK, OUT), lambda i: (i, 0)),
        ],
        out_specs=pl.BlockSpec((PBLK, OUT), lambda i: (i, 0)),
        out_shape=jax.ShapeDtypeStruct((NP, OUT), jnp.float32),
    )(o[0], o[1])

    return (out_full[:N], 0)


# speedup vs baseline: 87.7256x; 1.2642x over previous
import functools

import jax
import jax.numpy as jnp
from jax import lax
from jax.experimental import pallas as pl
from jax.experimental.pallas import tpu as pltpu
from jax.experimental.pallas import tpu_sc as plsc

NC = 2
NS = 16
NW = NC * NS
CHUNK = 128
NB = 8
LANES = 16


def _rsqrt16(x):
    i = plsc.bitcast(x, jnp.int32)
    i = jnp.int32(0x5F3759DF) - lax.shift_right_arithmetic(i, 1)
    y = plsc.bitcast(i, jnp.float32)
    xh = x * 0.5
    for _ in range(3):
        y = y * (1.5 - xh * y * y)
    return y


def _main_kernel(N, OUT, NCH, NPR):
    NP = NS * NPR
    TAIL = N - (NS - 1) * NPR
    QD = NCH // NS
    RD = NCH % NS
    QM = NCH // NW
    RM = NCH % NW
    mesh = plsc.VectorSubcoreMesh(core_axis_name="c", subcore_axis_name="s")

    @functools.partial(
        pl.kernel,
        out_type=jax.ShapeDtypeStruct((NC, NP, OUT), jnp.float32),
        mesh=mesh,
        scratch_types=[
            pltpu.VMEM((QD + 1, CHUNK), jnp.int32),
            pltpu.VMEM((QM + 1, CHUNK), jnp.int32),
            pltpu.VMEM((QM + 1, CHUNK), jnp.int32),
            pltpu.VMEM((CHUNK,), jnp.float32),
            pltpu.VMEM((NB, CHUNK, OUT), jnp.float32),
            pltpu.VMEM((NPR,), jnp.float32),
            pltpu.VMEM((NPR,), jnp.float32),
            pltpu.VMEM((NPR, OUT), jnp.float32),
            pltpu.VMEM((NPR, OUT), jnp.float32),
            pltpu.VMEM((OUT,), jnp.float32),
            pltpu.VMEM_SHARED((NP,), jnp.float32),
            pltpu.VMEM_SHARED((NP, OUT), jnp.float32),
            pltpu.VMEM_SHARED((NP, OUT), jnp.float32),
            [pltpu.SemaphoreType.DMA] * NB,
            [pltpu.SemaphoreType.DMA] * NB,
        ],
        compiler_params=pltpu.CompilerParams(
            use_tc_tiling_on_sc=False, needs_layout_passes=False
        ),
    )
    def main_k(h_hbm, ei_hbm, b_hbm, out_hbm, didx_v, msrc_v, mdst_v, ones_v,
               rows_v, degb, disb, hbuf, abuf, bbuf, deg_sp, g_sp, acc_sp,
               gsems, ssems):
        cid = lax.axis_index("c")
        sid = lax.axis_index("s")
        wid = cid * NS + sid
        base = sid * NPR

        ones16 = jnp.ones((LANES,), jnp.float32)
        zeros16 = jnp.zeros((LANES,), jnp.float32)
        for i in range(CHUNK // LANES):
            ones_v[pl.ds(i * LANES, LANES)] = ones16
        for i in range(NPR // LANES):
            degb[pl.ds(i * LANES, LANES)] = zeros16

        dbase = sid * QD
        dcnt = QD + (sid < RD).astype(jnp.int32)
        pltpu.sync_copy(ei_hbm.at[1, pl.ds(dbase, QD)], didx_v.at[pl.ds(0, QD)])

        @pl.when(sid < RD)
        def _():
            pltpu.sync_copy(
                ei_hbm.at[1, pl.ds(NS * QD + sid, 1)], didx_v.at[pl.ds(QD, 1)]
            )

        pltpu.sync_copy(degb, deg_sp.at[pl.ds(base, NPR)])
        plsc.subcore_barrier()

        def deg_grp(i, c):
            j0 = i * NB
            descs = [
                pltpu.async_copy(
                    ones_v, deg_sp.at[didx_v.at[j0 + k]], ssems[k], add=True
                )
                for k in range(NB)
            ]
            for d in descs:
                d.wait()
            return c

        lax.fori_loop(0, QD // NB, deg_grp, 0)
        for k in range(QD + 1 - (QD // NB) * NB):
            j = (QD // NB) * NB + k

            @pl.when(j < dcnt)
            def _():
                pltpu.sync_copy(ones_v, deg_sp.at[didx_v.at[j]], add=True)

        mbase = wid * QM
        mcnt = QM + (wid < RM).astype(jnp.int32)
        pltpu.sync_copy(ei_hbm.at[0, pl.ds(mbase, QM)], msrc_v.at[pl.ds(0, QM)])
        pltpu.sync_copy(ei_hbm.at[1, pl.ds(mbase, QM)], mdst_v.at[pl.ds(0, QM)])

        @pl.when(wid < RM)
        def _():
            pltpu.sync_copy(
                ei_hbm.at[0, pl.ds(NW * QM + wid, 1)], msrc_v.at[pl.ds(QM, 1)]
            )
            pltpu.sync_copy(
                ei_hbm.at[1, pl.ds(NW * QM + wid, 1)], mdst_v.at[pl.ds(QM, 1)]
            )

        pltpu.sync_copy(b_hbm, bbuf)
        plsc.subcore_barrier()

        pltpu.sync_copy(deg_sp.at[pl.ds(base, NPR)], degb)

        @pl.when(sid < NS - 1)
        def _():
            pltpu.sync_copy(h_hbm.at[pl.ds(base, NPR)], hbuf)

        @pl.when(sid == NS - 1)
        def _():
            pltpu.sync_copy(
                h_hbm.at[pl.ds(base, TAIL)], hbuf.at[pl.ds(0, TAIL)]
            )
            for i in range(TAIL, NPR):
                hbuf[i, :] = zeros16

        def dis_loop(i, c):
            off = pl.multiple_of(i * LANES, LANES)
            d = degb[pl.ds(off, LANES)] + 1.0
            disb[pl.ds(off, LANES)] = _rsqrt16(d)
            return c

        lax.fori_loop(0, NPR // LANES, dis_loop, 0)

        def scale_loop(i, c):
            off = pl.multiple_of(i * LANES, LANES)
            dvec = disb[pl.ds(off, LANES)]
            for r in range(LANES):
                hbuf[off + r, :] = hbuf[off + r, :] * dvec[r]
            return c

        lax.fori_loop(0, NPR // LANES, scale_loop, 0)
        pltpu.sync_copy(hbuf, g_sp.at[pl.ds(base, NPR)])
        pltpu.sync_copy(hbuf, acc_sp.at[pl.ds(base, NPR)])
        plsc.subcore_barrier()

        def msg_grp(i, c):
            j0 = i * NB
            gd = [
                pltpu.async_copy(
                    g_sp.at[msrc_v.at[j0 + k]], rows_v.at[k], gsems[k]
                )
                for k in range(NB)
            ]
            sd = []
            for k in range(NB):
                gd[k].wait()
                sd.append(
                    pltpu.async_copy(
                        rows_v.at[k], acc_sp.at[mdst_v.at[j0 + k]], ssems[k],
                        add=True,
                    )
                )
            for d in sd:
                d.wait()
            return c

        lax.fori_loop(0, QM // NB, msg_grp, 0)
        for k in range(QM + 1 - (QM // NB) * NB):
            j = (QM // NB) * NB + k

            @pl.when(j < mcnt)
            def _():
                pltpu.async_copy(
                    g_sp.at[msrc_v.at[j]], rows_v.at[0], gsems[0]
                ).wait()
                pltpu.sync_copy(
                    rows_v.at[0], acc_sp.at[mdst_v.at[j]], add=True
                )

        plsc.subcore_barrier()

        pltpu.sync_copy(acc_sp.at[pl.ds(base, NPR)], abuf)
        sel0 = lax.select(cid == 0, 1.0, 0.0)
        bvec = bbuf[...]

        def fin_loop(i, c):
            off = pl.multiple_of(i * LANES, LANES)
            dvec = disb[pl.ds(off, LANES)]
            for r in range(LANES):
                row = abuf[off + r, :] - (1.0 - sel0) * hbuf[off + r, :]
                abuf[off + r, :] = row * dvec[r] + sel0 * bvec
            return c

        lax.fori_loop(0, NPR // LANES, fin_loop, 0)
        pltpu.sync_copy(abuf, out_hbm.at[cid, pl.ds(base, NPR)])

    return main_k


def _pre_body(x_ref, w_ref, h_ref):
    h_ref[...] = jnp.dot(
        x_ref[...], w_ref[...], preferred_element_type=jnp.float32
    )


def _post_body(a_ref, o_ref):
    o_ref[...] = a_ref[0] + a_ref[1]


def kernel(x, edge_index, W, b):
    N, IN = x.shape
    OUT = W.shape[1]
    E = edge_index.shape[1]

    ei = edge_index.astype(jnp.int32)
    if E % CHUNK:
        pad = CHUNK - E % CHUNK
        ei = jnp.concatenate([ei, jnp.full((2, pad), N, jnp.int32)], axis=1)
    NCH = ei.shape[1] // CHUNK
    ei3 = ei.reshape(2, NCH, CHUNK)

    NPR = -(-(N + 1) // (NS * LANES)) * LANES
    NP = NS * NPR

    BLK = 2000 if N % 2000 == 0 else 8
    h = pl.pallas_call(
        _pre_body,
        grid=(N // BLK,),
        in_specs=[
            pl.BlockSpec((BLK, IN), lambda i: (i, 0)),
            pl.BlockSpec((IN, OUT), lambda i: (0, 0)),
        ],
        out_specs=pl.BlockSpec((BLK, OUT), lambda i: (i, 0)),
        out_shape=jax.ShapeDtypeStruct((N, OUT), jnp.float32),
    )(x, W)

    o = _main_kernel(N, OUT, NCH, NPR)(h, ei3, b)

    PBLK = 2000 if N % 2000 == 0 else 8
    out_full = pl.pallas_call(
        _post_body,
        grid=(N // PBLK,),
        in_specs=[pl.BlockSpec((NC, PBLK, OUT), lambda i: (0, i, 0))],
        out_specs=pl.BlockSpec((PBLK, OUT), lambda i: (i, 0)),
        out_shape=jax.ShapeDtypeStruct((N, OUT), jnp.float32),
    )(o)

    return (out_full, 0)


# speedup vs baseline: 90.7901x; 1.0349x over previous
import functools

import jax
import jax.numpy as jnp
from jax import lax
from jax.experimental import pallas as pl
from jax.experimental.pallas import tpu as pltpu
from jax.experimental.pallas import tpu_sc as plsc

NC = 2
NS = 16
NW = NC * NS
CHUNK = 128
NB = 8
LANES = 16


def _rsqrt16(x):
    i = plsc.bitcast(x, jnp.int32)
    i = jnp.int32(0x5F3759DF) - lax.shift_right_arithmetic(i, 1)
    y = plsc.bitcast(i, jnp.float32)
    xh = x * 0.5
    for _ in range(3):
        y = y * (1.5 - xh * y * y)
    return y


def _main_kernel(N, OUT, NCH, NPR):
    NP = NS * NPR
    TAIL = N - (NS - 1) * NPR
    M2 = -(-(-(-NCH // NS)) // 8) * 8
    F2 = NCH // M2
    REM2 = NCH - F2 * M2
    M1 = -(-(-(-NCH // NW)) // 8) * 8
    F1 = NCH // M1
    REM1 = NCH - F1 * M1
    mesh = plsc.VectorSubcoreMesh(core_axis_name="c", subcore_axis_name="s")

    @functools.partial(
        pl.kernel,
        out_type=jax.ShapeDtypeStruct((NC, NP, OUT), jnp.float32),
        mesh=mesh,
        scratch_types=[
            pltpu.VMEM((M2, CHUNK), jnp.int32),
            pltpu.VMEM((M1, CHUNK), jnp.int32),
            pltpu.VMEM((M1, CHUNK), jnp.int32),
            pltpu.VMEM((CHUNK,), jnp.float32),
            pltpu.VMEM((NB, CHUNK, OUT), jnp.float32),
            pltpu.VMEM((NPR,), jnp.float32),
            pltpu.VMEM((NPR,), jnp.float32),
            pltpu.VMEM((NPR, OUT), jnp.float32),
            pltpu.VMEM((NPR, OUT), jnp.float32),
            pltpu.VMEM((OUT,), jnp.float32),
            pltpu.VMEM_SHARED((NP,), jnp.float32),
            pltpu.VMEM_SHARED((NP, OUT), jnp.float32),
            pltpu.VMEM_SHARED((NP, OUT), jnp.float32),
            [pltpu.SemaphoreType.DMA] * NB,
            [pltpu.SemaphoreType.DMA] * NB,
        ],
        compiler_params=pltpu.CompilerParams(
            use_tc_tiling_on_sc=False, needs_layout_passes=False
        ),
    )
    def main_k(h_hbm, ei_hbm, b_hbm, out_hbm, didx_v, msrc_v, mdst_v, ones_v,
               rows_v, degb, disb, hbuf, abuf, bbuf, deg_sp, g_sp, acc_sp,
               gsems, ssems):
        cid = lax.axis_index("c")
        sid = lax.axis_index("s")
        wid = cid * NS + sid
        base = sid * NPR

        ones16 = jnp.ones((LANES,), jnp.float32)
        zeros16 = jnp.zeros((LANES,), jnp.float32)
        for i in range(CHUNK // LANES):
            ones_v[pl.ds(i * LANES, LANES)] = ones16
        for i in range(NPR // LANES):
            degb[pl.ds(i * LANES, LANES)] = zeros16

        def emit_deg(cnt):
            G, T = cnt // NB, cnt % NB
            if G > 0:
                def agrp(i, c):
                    j0 = i * NB
                    for k in range(NB):
                        pltpu.async_copy(
                            ones_v, deg_sp.at[didx_v.at[j0 + k]], ssems[k],
                            add=True,
                        )

                    @pl.when(i > 0)
                    def _():
                        for k in range(NB):
                            pltpu.make_async_copy(
                                ones_v, deg_sp.at[didx_v.at[j0 - NB + k]],
                                ssems[k],
                            ).wait()

                    return c

                lax.fori_loop(0, G, agrp, 0)
                for k in range(NB):
                    pltpu.make_async_copy(
                        ones_v, deg_sp.at[didx_v.at[(G - 1) * NB + k]],
                        ssems[k],
                    ).wait()
            for t in range(T):
                pltpu.sync_copy(
                    ones_v, deg_sp.at[didx_v.at[G * NB + t]], add=True
                )

        if F2 > 0:
            @pl.when(sid < F2)
            def _():
                pltpu.sync_copy(ei_hbm.at[1, pl.ds(sid * M2, M2)], didx_v)

        if REM2 > 0:
            @pl.when(sid == F2)
            def _():
                pltpu.sync_copy(
                    ei_hbm.at[1, pl.ds(F2 * M2, REM2)],
                    didx_v.at[pl.ds(0, REM2)],
                )

        pltpu.sync_copy(degb, deg_sp.at[pl.ds(base, NPR)])
        plsc.subcore_barrier()

        if F2 > 0:
            @pl.when(sid < F2)
            def _():
                emit_deg(M2)

        if REM2 > 0:
            @pl.when(sid == F2)
            def _():
                emit_deg(REM2)

        if F1 > 0:
            @pl.when(wid < F1)
            def _():
                pltpu.sync_copy(ei_hbm.at[0, pl.ds(wid * M1, M1)], msrc_v)
                pltpu.sync_copy(ei_hbm.at[1, pl.ds(wid * M1, M1)], mdst_v)

        if REM1 > 0:
            @pl.when(wid == F1)
            def _():
                pltpu.sync_copy(
                    ei_hbm.at[0, pl.ds(F1 * M1, REM1)],
                    msrc_v.at[pl.ds(0, REM1)],
                )
                pltpu.sync_copy(
                    ei_hbm.at[1, pl.ds(F1 * M1, REM1)],
                    mdst_v.at[pl.ds(0, REM1)],
                )

        pltpu.sync_copy(b_hbm, bbuf)
        plsc.subcore_barrier()

        pltpu.sync_copy(deg_sp.at[pl.ds(base, NPR)], degb)

        @pl.when(sid < NS - 1)
        def _():
            pltpu.sync_copy(h_hbm.at[pl.ds(base, NPR)], hbuf)

        @pl.when(sid == NS - 1)
        def _():
            pltpu.sync_copy(
                h_hbm.at[pl.ds(base, TAIL)], hbuf.at[pl.ds(0, TAIL)]
            )
            for i in range(TAIL, NPR):
                hbuf[i, :] = zeros16

        def dis_loop(i, c):
            off = pl.multiple_of(i * LANES, LANES)
            d = degb[pl.ds(off, LANES)] + 1.0
            disb[pl.ds(off, LANES)] = _rsqrt16(d)
            return c

        lax.fori_loop(0, NPR // LANES, dis_loop, 0)

        def scale_loop(i, c):
            off = pl.multiple_of(i * LANES, LANES)
            dvec = disb[pl.ds(off, LANES)]
            for r in range(LANES):
                hbuf[off + r, :] = hbuf[off + r, :] * dvec[r]
            return c

        lax.fori_loop(0, NPR // LANES, scale_loop, 0)
        pltpu.sync_copy(hbuf, g_sp.at[pl.ds(base, NPR)])
        pltpu.sync_copy(hbuf, acc_sp.at[pl.ds(base, NPR)])
        plsc.subcore_barrier()

        def emit_msg(cnt):
            G, T = cnt // NB, cnt % NB
            if G > 0:
                def mgrp(i, c):
                    j0 = i * NB

                    @pl.when(i > 0)
                    def _():
                        for k in range(NB):
                            pltpu.make_async_copy(
                                rows_v.at[k],
                                acc_sp.at[mdst_v.at[j0 - NB + k]],
                                ssems[k],
                            ).wait()

                    gd = [
                        pltpu.async_copy(
                            g_sp.at[msrc_v.at[j0 + k]], rows_v.at[k], gsems[k]
                        )
                        for k in range(NB)
                    ]
                    for k in range(NB):
                        gd[k].wait()
                        pltpu.async_copy(
                            rows_v.at[k], acc_sp.at[mdst_v.at[j0 + k]],
                            ssems[k], add=True,
                        )
                    return c

                lax.fori_loop(0, G, mgrp, 0)
                for k in range(NB):
                    pltpu.make_async_copy(
                        rows_v.at[k], acc_sp.at[mdst_v.at[(G - 1) * NB + k]],
                        ssems[k],
                    ).wait()
            for t in range(T):
                j = G * NB + t
                pltpu.async_copy(
                    g_sp.at[msrc_v.at[j]], rows_v.at[0], gsems[0]
                ).wait()
                pltpu.sync_copy(rows_v.at[0], acc_sp.at[mdst_v.at[j]], add=True)

        if F1 > 0:
            @pl.when(wid < F1)
            def _():
                emit_msg(M1)

        if REM1 > 0:
            @pl.when(wid == F1)
            def _():
                emit_msg(REM1)

        plsc.subcore_barrier()

        pltpu.sync_copy(acc_sp.at[pl.ds(base, NPR)], abuf)
        sel0 = lax.select(cid == 0, 1.0, 0.0)
        bvec = bbuf[...]

        def fin_loop(i, c):
            off = pl.multiple_of(i * LANES, LANES)
            dvec = disb[pl.ds(off, LANES)]
            for r in range(LANES):
                row = abuf[off + r, :] - (1.0 - sel0) * hbuf[off + r, :]
                abuf[off + r, :] = row * dvec[r] + sel0 * bvec
            return c

        lax.fori_loop(0, NPR // LANES, fin_loop, 0)
        pltpu.sync_copy(abuf, out_hbm.at[cid, pl.ds(base, NPR)])

    return main_k


def _pre_body(x_ref, w_ref, h_ref):
    h_ref[...] = jnp.dot(
        x_ref[...], w_ref[...], preferred_element_type=jnp.float32
    )


def _post_body(a_ref, o_ref):
    o_ref[...] = a_ref[0] + a_ref[1]


def kernel(x, edge_index, W, b):
    N, IN = x.shape
    OUT = W.shape[1]
    E = edge_index.shape[1]

    ei = edge_index.astype(jnp.int32)
    if E % CHUNK:
        pad = CHUNK - E % CHUNK
        ei = jnp.concatenate([ei, jnp.full((2, pad), N, jnp.int32)], axis=1)
    NCH = ei.shape[1] // CHUNK
    ei3 = ei.reshape(2, NCH, CHUNK)

    NPR = -(-(N + 1) // (NS * LANES)) * LANES
    NP = NS * NPR

    BLK = 2000 if N % 2000 == 0 else 8
    h = pl.pallas_call(
        _pre_body,
        grid=(N // BLK,),
        in_specs=[
            pl.BlockSpec((BLK, IN), lambda i: (i, 0)),
            pl.BlockSpec((IN, OUT), lambda i: (0, 0)),
        ],
        out_specs=pl.BlockSpec((BLK, OUT), lambda i: (i, 0)),
        out_shape=jax.ShapeDtypeStruct((N, OUT), jnp.float32),
    )(x, W)

    o = _main_kernel(N, OUT, NCH, NPR)(h, ei3, b)

    PBLK = 2000 if N % 2000 == 0 else 8
    out_full = pl.pallas_call(
        _post_body,
        grid=(N // PBLK,),
        in_specs=[pl.BlockSpec((NC, PBLK, OUT), lambda i: (0, i, 0))],
        out_specs=pl.BlockSpec((PBLK, OUT), lambda i: (i, 0)),
        out_shape=jax.ShapeDtypeStruct((N, OUT), jnp.float32),
    )(o)

    return (out_full, 0)


# speedup vs baseline: 98.9391x; 1.0898x over previous
import functools

import jax
import jax.numpy as jnp
from jax import lax
from jax.experimental import pallas as pl
from jax.experimental.pallas import tpu as pltpu
from jax.experimental.pallas import tpu_sc as plsc

NC = 2
NS = 16
NW = NC * NS
CHUNK = 128
NB = 8
LANES = 16


def _rsqrt16(x):
    i = plsc.bitcast(x, jnp.int32)
    i = jnp.int32(0x5F3759DF) - lax.shift_right_arithmetic(i, 1)
    y = plsc.bitcast(i, jnp.float32)
    xh = x * 0.5
    for _ in range(3):
        y = y * (1.5 - xh * y * y)
    return y


def _main_kernel(N, OUT, NCH, NPR):
    NP = NS * NPR
    TAIL = N - (NS - 1) * NPR
    M2 = -(-(-(-NCH // NS)) // 8) * 8
    F2 = NCH // M2
    REM2 = NCH - F2 * M2
    M1 = -(-(-(-NCH // NW)) // 8) * 8
    F1 = NCH // M1
    REM1 = NCH - F1 * M1
    mesh = plsc.VectorSubcoreMesh(core_axis_name="c", subcore_axis_name="s")

    NPO = NP * OUT // 128

    @functools.partial(
        pl.kernel,
        out_type=jax.ShapeDtypeStruct((NC, NPO, 128), jnp.float32),
        mesh=mesh,
        scratch_types=[
            pltpu.VMEM((M2, CHUNK), jnp.int32),
            pltpu.VMEM((M1, CHUNK), jnp.int32),
            pltpu.VMEM((M1, CHUNK), jnp.int32),
            pltpu.VMEM((CHUNK,), jnp.float32),
            pltpu.VMEM((NB, CHUNK, OUT), jnp.float32),
            pltpu.VMEM((NPR,), jnp.float32),
            pltpu.VMEM((NPR,), jnp.float32),
            pltpu.VMEM((NPR, OUT), jnp.float32),
            pltpu.VMEM((NPR, OUT), jnp.float32),
            pltpu.VMEM((NPR * OUT // 128, 128), jnp.float32),
            pltpu.VMEM((OUT,), jnp.float32),
            pltpu.VMEM_SHARED((NP,), jnp.float32),
            pltpu.VMEM_SHARED((NP, OUT), jnp.float32),
            pltpu.VMEM_SHARED((NP, OUT), jnp.float32),
            [pltpu.SemaphoreType.DMA] * NB,
            [pltpu.SemaphoreType.DMA] * NB,
        ],
        compiler_params=pltpu.CompilerParams(
            use_tc_tiling_on_sc=False, needs_layout_passes=False
        ),
    )
    def main_k(h_hbm, ei_hbm, b_hbm, out_hbm, didx_v, msrc_v, mdst_v, ones_v,
               rows_v, degb, disb, hbuf, abuf, obuf, bbuf, deg_sp, g_sp,
               acc_sp, gsems, ssems):
        cid = lax.axis_index("c")
        sid = lax.axis_index("s")
        wid = cid * NS + sid
        base = sid * NPR

        ones16 = jnp.ones((LANES,), jnp.float32)
        zeros16 = jnp.zeros((LANES,), jnp.float32)
        for i in range(CHUNK // LANES):
            ones_v[pl.ds(i * LANES, LANES)] = ones16
        for i in range(NPR // LANES):
            degb[pl.ds(i * LANES, LANES)] = zeros16

        def emit_deg(cnt):
            G, T = cnt // NB, cnt % NB
            if G > 0:
                def agrp(i, c):
                    j0 = i * NB
                    for k in range(NB):
                        pltpu.async_copy(
                            ones_v, deg_sp.at[didx_v.at[j0 + k]], ssems[k],
                            add=True,
                        )

                    @pl.when(i > 0)
                    def _():
                        for k in range(NB):
                            pltpu.make_async_copy(
                                ones_v, deg_sp.at[didx_v.at[j0 - NB + k]],
                                ssems[k],
                            ).wait()

                    return c

                lax.fori_loop(0, G, agrp, 0)
                for k in range(NB):
                    pltpu.make_async_copy(
                        ones_v, deg_sp.at[didx_v.at[(G - 1) * NB + k]],
                        ssems[k],
                    ).wait()
            for t in range(T):
                pltpu.sync_copy(
                    ones_v, deg_sp.at[didx_v.at[G * NB + t]], add=True
                )

        if F2 > 0:
            @pl.when(sid < F2)
            def _():
                pltpu.sync_copy(ei_hbm.at[1, pl.ds(sid * M2, M2)], didx_v)

        if REM2 > 0:
            @pl.when(sid == F2)
            def _():
                pltpu.sync_copy(
                    ei_hbm.at[1, pl.ds(F2 * M2, REM2)],
                    didx_v.at[pl.ds(0, REM2)],
                )

        pltpu.sync_copy(degb, deg_sp.at[pl.ds(base, NPR)])
        plsc.subcore_barrier()

        if F2 > 0:
            @pl.when(sid < F2)
            def _():
                emit_deg(M2)

        if REM2 > 0:
            @pl.when(sid == F2)
            def _():
                emit_deg(REM2)

        if F1 > 0:
            @pl.when(wid < F1)
            def _():
                pltpu.sync_copy(ei_hbm.at[0, pl.ds(wid * M1, M1)], msrc_v)
                pltpu.sync_copy(ei_hbm.at[1, pl.ds(wid * M1, M1)], mdst_v)

        if REM1 > 0:
            @pl.when(wid == F1)
            def _():
                pltpu.sync_copy(
                    ei_hbm.at[0, pl.ds(F1 * M1, REM1)],
                    msrc_v.at[pl.ds(0, REM1)],
                )
                pltpu.sync_copy(
                    ei_hbm.at[1, pl.ds(F1 * M1, REM1)],
                    mdst_v.at[pl.ds(0, REM1)],
                )

        pltpu.sync_copy(b_hbm, bbuf)
        plsc.subcore_barrier()

        pltpu.sync_copy(deg_sp.at[pl.ds(base, NPR)], degb)

        @pl.when(sid < NS - 1)
        def _():
            pltpu.sync_copy(h_hbm.at[pl.ds(base, NPR)], hbuf)

        @pl.when(sid == NS - 1)
        def _():
            pltpu.sync_copy(
                h_hbm.at[pl.ds(base, TAIL)], hbuf.at[pl.ds(0, TAIL)]
            )
            for i in range(TAIL, NPR):
                hbuf[i, :] = zeros16

        def dis_loop(i, c):
            off = pl.multiple_of(i * LANES, LANES)
            d = degb[pl.ds(off, LANES)] + 1.0
            disb[pl.ds(off, LANES)] = _rsqrt16(d)
            return c

        lax.fori_loop(0, NPR // LANES, dis_loop, 0)

        def scale_loop(i, c):
            off = pl.multiple_of(i * LANES, LANES)
            dvec = disb[pl.ds(off, LANES)]
            for r in range(LANES):
                hbuf[off + r, :] = hbuf[off + r, :] * dvec[r]
            return c

        lax.fori_loop(0, NPR // LANES, scale_loop, 0)
        pltpu.sync_copy(hbuf, g_sp.at[pl.ds(base, NPR)])
        pltpu.sync_copy(hbuf, acc_sp.at[pl.ds(base, NPR)])
        plsc.subcore_barrier()

        def emit_msg(cnt):
            G, T = cnt // NB, cnt % NB
            if G > 0:
                def mgrp(i, c):
                    j0 = i * NB

                    @pl.when(i > 0)
                    def _():
                        for k in range(NB):
                            pltpu.make_async_copy(
                                rows_v.at[k],
                                acc_sp.at[mdst_v.at[j0 - NB + k]],
                                ssems[k],
                            ).wait()

                    gd = [
                        pltpu.async_copy(
                            g_sp.at[msrc_v.at[j0 + k]], rows_v.at[k], gsems[k]
                        )
                        for k in range(NB)
                    ]
                    for k in range(NB):
                        gd[k].wait()
                        pltpu.async_copy(
                            rows_v.at[k], acc_sp.at[mdst_v.at[j0 + k]],
                            ssems[k], add=True,
                        )
                    return c

                lax.fori_loop(0, G, mgrp, 0)
                for k in range(NB):
                    pltpu.make_async_copy(
                        rows_v.at[k], acc_sp.at[mdst_v.at[(G - 1) * NB + k]],
                        ssems[k],
                    ).wait()
            for t in range(T):
                j = G * NB + t
                pltpu.async_copy(
                    g_sp.at[msrc_v.at[j]], rows_v.at[0], gsems[0]
                ).wait()
                pltpu.sync_copy(rows_v.at[0], acc_sp.at[mdst_v.at[j]], add=True)

        if F1 > 0:
            @pl.when(wid < F1)
            def _():
                emit_msg(M1)

        if REM1 > 0:
            @pl.when(wid == F1)
            def _():
                emit_msg(REM1)

        plsc.subcore_barrier()

        pltpu.sync_copy(acc_sp.at[pl.ds(base, NPR)], abuf)
        sel0 = lax.select(cid == 0, 1.0, 0.0)
        bvec = bbuf[...]

        ROWS_PER_128 = 128 // OUT

        def fin_loop(i, c):
            off = pl.multiple_of(i * LANES, LANES)
            dvec = disb[pl.ds(off, LANES)]
            for r in range(LANES):
                row = abuf[off + r, :] - (1.0 - sel0) * hbuf[off + r, :]
                q = (LANES // ROWS_PER_128) * i + r // ROWS_PER_128
                obuf[q, pl.ds((r % ROWS_PER_128) * OUT, OUT)] = (
                    row * dvec[r] + sel0 * bvec
                )
            return c

        lax.fori_loop(0, NPR // LANES, fin_loop, 0)
        TPO = NPR * OUT // 128
        pltpu.sync_copy(obuf, out_hbm.at[cid, pl.ds(sid * TPO, TPO)])

    return main_k


def _pre_body(x_ref, w_ref, h_ref):
    h_ref[...] = jnp.dot(
        x_ref[...], w_ref[...], preferred_element_type=jnp.float32
    )


def _post_body(a_ref, o_ref):
    o_ref[...] = a_ref[0] + a_ref[1]


def kernel(x, edge_index, W, b):
    N, IN = x.shape
    OUT = W.shape[1]
    E = edge_index.shape[1]

    ei = edge_index.astype(jnp.int32)
    if E % CHUNK:
        pad = CHUNK - E % CHUNK
        ei = jnp.concatenate([ei, jnp.full((2, pad), N, jnp.int32)], axis=1)
    NCH = ei.shape[1] // CHUNK
    ei3 = ei.reshape(2, NCH, CHUNK)

    NPR = -(-(N + 1) // (NS * LANES)) * LANES
    NP = NS * NPR

    BLK = 2000 if N % 2000 == 0 else 8
    h = pl.pallas_call(
        _pre_body,
        grid=(N // BLK,),
        in_specs=[
            pl.BlockSpec((BLK, IN), lambda i: (i, 0)),
            pl.BlockSpec((IN, OUT), lambda i: (0, 0)),
        ],
        out_specs=pl.BlockSpec((BLK, OUT), lambda i: (i, 0)),
        out_shape=jax.ShapeDtypeStruct((N, OUT), jnp.float32),
    )(x, W)

    o = _main_kernel(N, OUT, NCH, NPR)(h, ei3, b)

    NPO = NP * OUT // 128
    PBLK = 256 if NPO % 256 == 0 else 8
    out128 = pl.pallas_call(
        _post_body,
        grid=(NPO // PBLK,),
        in_specs=[pl.BlockSpec((NC, PBLK, 128), lambda i: (0, i, 0))],
        out_specs=pl.BlockSpec((PBLK, 128), lambda i: (i, 0)),
        out_shape=jax.ShapeDtypeStruct((NPO, 128), jnp.float32),
    )(o)

    return (out128.reshape(NP, OUT)[:N], 0)


# speedup vs baseline: 98.9627x; 1.0002x over previous
import functools

import jax
import jax.numpy as jnp
from jax import lax
from jax.experimental import pallas as pl
from jax.experimental.pallas import tpu as pltpu
from jax.experimental.pallas import tpu_sc as plsc

NC = 2
NS = 16
NW = NC * NS
CHUNK = 128
NB = 8
LANES = 16


def _rsqrt16(x):
    i = plsc.bitcast(x, jnp.int32)
    i = jnp.int32(0x5F3759DF) - lax.shift_right_arithmetic(i, 1)
    y = plsc.bitcast(i, jnp.float32)
    xh = x * 0.5
    for _ in range(3):
        y = y * (1.5 - xh * y * y)
    return y


def _main_kernel(N, OUT, NCH, NPR):
    NP = NS * NPR
    TAIL = N - (NS - 1) * NPR
    M2 = -(-(-(-NCH // NS)) // 8) * 8
    F2 = NCH // M2
    REM2 = NCH - F2 * M2
    M1 = -(-(-(-NCH // NW)) // 8) * 8
    F1 = NCH // M1
    REM1 = NCH - F1 * M1
    mesh = plsc.VectorSubcoreMesh(core_axis_name="c", subcore_axis_name="s")

    NPO = NP * OUT // 128

    @functools.partial(
        pl.kernel,
        out_type=jax.ShapeDtypeStruct((NC, NPO, 128), jnp.float32),
        mesh=mesh,
        scratch_types=[
            pltpu.VMEM((M2, CHUNK), jnp.int32),
            pltpu.VMEM((M1, CHUNK), jnp.int32),
            pltpu.VMEM((M1, CHUNK), jnp.int32),
            pltpu.VMEM((CHUNK,), jnp.float32),
            pltpu.VMEM((NB, CHUNK, OUT), jnp.float32),
            pltpu.VMEM((NPR,), jnp.float32),
            pltpu.VMEM((NPR,), jnp.float32),
            pltpu.VMEM((NPR, OUT), jnp.float32),
            pltpu.VMEM((NPR, OUT), jnp.float32),
            pltpu.VMEM((NPR, OUT), jnp.float32),
            pltpu.VMEM((NPR * OUT // 128, 128), jnp.float32),
            pltpu.VMEM((OUT,), jnp.float32),
            pltpu.VMEM_SHARED((NP,), jnp.float32),
            pltpu.VMEM_SHARED((NP, OUT), jnp.float32),
            pltpu.VMEM_SHARED((NP, OUT), jnp.float32),
            [pltpu.SemaphoreType.DMA] * NB,
            [pltpu.SemaphoreType.DMA] * NB,
        ],
        compiler_params=pltpu.CompilerParams(
            use_tc_tiling_on_sc=False, needs_layout_passes=False
        ),
    )
    def main_k(h_hbm, ei_hbm, b_hbm, out_hbm, didx_v, msrc_v, mdst_v, ones_v,
               rows_v, degb, disb, hbuf, gslice, abuf, obuf, bbuf, deg_sp,
               g_sp, acc_sp, gsems, ssems):
        cid = lax.axis_index("c")
        sid = lax.axis_index("s")
        wid = cid * NS + sid
        base = sid * NPR

        ones16 = jnp.ones((LANES,), jnp.float32)
        zeros16 = jnp.zeros((LANES,), jnp.float32)
        for i in range(CHUNK // LANES):
            ones_v[pl.ds(i * LANES, LANES)] = ones16
        for i in range(NPR // LANES):
            degb[pl.ds(i * LANES, LANES)] = zeros16

        def emit_deg(cnt):
            G, T = cnt // NB, cnt % NB
            if G > 0:
                def agrp(i, c):
                    j0 = i * NB
                    for k in range(NB):
                        pltpu.async_copy(
                            ones_v, deg_sp.at[didx_v.at[j0 + k]], ssems[k],
                            add=True,
                        )

                    @pl.when(i > 0)
                    def _():
                        for k in range(NB):
                            pltpu.make_async_copy(
                                ones_v, deg_sp.at[didx_v.at[j0 - NB + k]],
                                ssems[k],
                            ).wait()

                    return c

                lax.fori_loop(0, G, agrp, 0)
                for k in range(NB):
                    pltpu.make_async_copy(
                        ones_v, deg_sp.at[didx_v.at[(G - 1) * NB + k]],
                        ssems[k],
                    ).wait()
            for t in range(T):
                pltpu.sync_copy(
                    ones_v, deg_sp.at[didx_v.at[G * NB + t]], add=True
                )

        if F2 > 0:
            @pl.when(sid < F2)
            def _():
                pltpu.sync_copy(ei_hbm.at[1, pl.ds(sid * M2, M2)], didx_v)

        if REM2 > 0:
            @pl.when(sid == F2)
            def _():
                pltpu.sync_copy(
                    ei_hbm.at[1, pl.ds(F2 * M2, REM2)],
                    didx_v.at[pl.ds(0, REM2)],
                )

        pltpu.sync_copy(degb, deg_sp.at[pl.ds(base, NPR)])
        plsc.subcore_barrier()

        if F2 > 0:
            @pl.when(sid < F2)
            def _():
                emit_deg(M2)

        if REM2 > 0:
            @pl.when(sid == F2)
            def _():
                emit_deg(REM2)

        if F1 > 0:
            @pl.when(wid < F1)
            def _():
                pltpu.sync_copy(ei_hbm.at[0, pl.ds(wid * M1, M1)], msrc_v)
                pltpu.sync_copy(ei_hbm.at[1, pl.ds(wid * M1, M1)], mdst_v)

        if REM1 > 0:
            @pl.when(wid == F1)
            def _():
                pltpu.sync_copy(
                    ei_hbm.at[0, pl.ds(F1 * M1, REM1)],
                    msrc_v.at[pl.ds(0, REM1)],
                )
                pltpu.sync_copy(
                    ei_hbm.at[1, pl.ds(F1 * M1, REM1)],
                    mdst_v.at[pl.ds(0, REM1)],
                )

        pltpu.sync_copy(b_hbm, bbuf)
        plsc.subcore_barrier()

        pltpu.sync_copy(deg_sp.at[pl.ds(base, NPR)], degb)

        @pl.when(sid < NS - 1)
        def _():
            pltpu.sync_copy(h_hbm.at[pl.ds(base, NPR)], hbuf)

        @pl.when(sid == NS - 1)
        def _():
            pltpu.sync_copy(
                h_hbm.at[pl.ds(base, TAIL)], hbuf.at[pl.ds(0, TAIL)]
            )
            for i in range(TAIL, NPR):
                hbuf[i, :] = zeros16

        def dis_loop(i, c):
            off = pl.multiple_of(i * LANES, LANES)
            d = degb[pl.ds(off, LANES)] + 1.0
            disb[pl.ds(off, LANES)] = _rsqrt16(d)
            return c

        lax.fori_loop(0, NPR // LANES, dis_loop, 0)

        def scale_loop(i, c):
            off = pl.multiple_of(i * LANES, LANES)
            dvec = disb[pl.ds(off, LANES)]
            for r in range(LANES):
                gslice[off + r, :] = hbuf[off + r, :] * dvec[r]
            return c

        lax.fori_loop(0, NPR // LANES, scale_loop, 0)
        pltpu.sync_copy(gslice, g_sp.at[pl.ds(base, NPR)])
        pltpu.sync_copy(gslice, acc_sp.at[pl.ds(base, NPR)])
        plsc.subcore_barrier()

        def emit_msg(cnt):
            G, T = cnt // NB, cnt % NB
            if G > 0:
                def mgrp(i, c):
                    j0 = i * NB

                    @pl.when(i > 0)
                    def _():
                        for k in range(NB):
                            pltpu.make_async_copy(
                                rows_v.at[k],
                                acc_sp.at[mdst_v.at[j0 - NB + k]],
                                ssems[k],
                            ).wait()

                    gd = [
                        pltpu.async_copy(
                            g_sp.at[msrc_v.at[j0 + k]], rows_v.at[k], gsems[k]
                        )
                        for k in range(NB)
                    ]
                    for k in range(NB):
                        gd[k].wait()
                        pltpu.async_copy(
                            rows_v.at[k], acc_sp.at[mdst_v.at[j0 + k]],
                            ssems[k], add=True,
                        )
                    return c

                lax.fori_loop(0, G, mgrp, 0)
                for k in range(NB):
                    pltpu.make_async_copy(
                        rows_v.at[k], acc_sp.at[mdst_v.at[(G - 1) * NB + k]],
                        ssems[k],
                    ).wait()
            for t in range(T):
                j = G * NB + t
                pltpu.async_copy(
                    g_sp.at[msrc_v.at[j]], rows_v.at[0], gsems[0]
                ).wait()
                pltpu.sync_copy(rows_v.at[0], acc_sp.at[mdst_v.at[j]], add=True)

        if F1 > 0:
            @pl.when(wid < F1)
            def _():
                emit_msg(M1)

        if REM1 > 0:
            @pl.when(wid == F1)
            def _():
                emit_msg(REM1)

        plsc.subcore_barrier()

        pltpu.sync_copy(acc_sp.at[pl.ds(base, NPR)], abuf)
        sel0 = lax.select(cid == 0, 1.0, 0.0)
        bvec = bbuf[...]

        ROWS_PER_128 = 128 // OUT

        def fin_loop(i, c):
            off = pl.multiple_of(i * LANES, LANES)
            dvec = disb[pl.ds(off, LANES)]
            for r in range(LANES):
                row = abuf[off + r, :] - (1.0 - sel0) * gslice[off + r, :]
                q = (LANES // ROWS_PER_128) * i + r // ROWS_PER_128
                obuf[q, pl.ds((r % ROWS_PER_128) * OUT, OUT)] = (
                    row * dvec[r] + sel0 * bvec
                )
            return c

        lax.fori_loop(0, NPR // LANES, fin_loop, 0)
        TPO = NPR * OUT // 128
        pltpu.sync_copy(obuf, out_hbm.at[cid, pl.ds(sid * TPO, TPO)])

    return main_k


def _pre_body(x_ref, w_ref, h_ref):
    h_ref[...] = jnp.dot(
        x_ref[...], w_ref[...], preferred_element_type=jnp.float32
    )


def _post_body(a_ref, o_ref):
    o_ref[...] = a_ref[0] + a_ref[1]


def kernel(x, edge_index, W, b):
    N, IN = x.shape
    OUT = W.shape[1]
    E = edge_index.shape[1]

    ei = edge_index.astype(jnp.int32)
    if E % CHUNK:
        pad = CHUNK - E % CHUNK
        ei = jnp.concatenate([ei, jnp.full((2, pad), N, jnp.int32)], axis=1)
    NCH = ei.shape[1] // CHUNK
    ei3 = ei.reshape(2, NCH, CHUNK)

    NPR = -(-(N + 1) // (NS * LANES)) * LANES
    NP = NS * NPR

    BLK = 2000 if N % 2000 == 0 else 8
    h = pl.pallas_call(
        _pre_body,
        grid=(N // BLK,),
        in_specs=[
            pl.BlockSpec((BLK, IN), lambda i: (i, 0)),
            pl.BlockSpec((IN, OUT), lambda i: (0, 0)),
        ],
        out_specs=pl.BlockSpec((BLK, OUT), lambda i: (i, 0)),
        out_shape=jax.ShapeDtypeStruct((N, OUT), jnp.float32),
    )(x, W)

    o = _main_kernel(N, OUT, NCH, NPR)(h, ei3, b)

    NPO = NP * OUT // 128
    PBLK = 256 if NPO % 256 == 0 else 8
    out128 = pl.pallas_call(
        _post_body,
        grid=(NPO // PBLK,),
        in_specs=[pl.BlockSpec((NC, PBLK, 128), lambda i: (0, i, 0))],
        out_specs=pl.BlockSpec((PBLK, 128), lambda i: (i, 0)),
        out_shape=jax.ShapeDtypeStruct((NPO, 128), jnp.float32),
    )(o)

    return (out128.reshape(NP, OUT)[:N], 0)
